# baseline (speedup 1.0000x reference)
import functools

import jax
import jax.numpy as jnp
from jax import lax
from jax.experimental import pallas as pl
from jax.experimental.pallas import tpu as pltpu
from jax.experimental.pallas import tpu_sc as plsc

_B, _N, _IN_DIM, _OUT_DIM = 2, 2048, 128, 128
_TOP_K = 4
_BR = 512

_NC, _NS, _L = 2, 16, 16
_NW = _NC * _NS
_NODES_W = (_B * _N) // _NW
_ROWS_W = _NODES_W * _TOP_K
_NCHUNK = 4
_NODES_C = _NODES_W // _NCHUNK
_ROWS_C = _ROWS_W // _NCHUNK


def _topk_body(pos_blk, pos_t, x_blk, wt_ref, b_ref,
               gidx_ref, wexp_ref, xw_ref):
    pi = pos_blk[0]
    pj = pos_t[0]

    xw_ref[0] = jnp.dot(x_blk[0], wt_ref[...],
                        preferred_element_type=jnp.float32) + b_ref[...]

    dx = pi[:, 0:1] - pj[0:1, :]
    dy = pi[:, 1:2] - pj[1:2, :]
    dz = pi[:, 2:3] - pj[2:3, :]
    dsq = dx * dx + dy * dy + dz * dz
    dist = jnp.sqrt(dsq + 1e-8)
    adj = jnp.exp(-(dist * dist) * 0.5)

    def _emit(idxs, ws):
        s = ((ws[0] + ws[1]) + ws[2]) + ws[3] + 1e-8
        boff = pl.program_id(0) * _N
        gidx_ref[0] = jnp.concatenate([i + boff for i in idxs], axis=1)
        wexp_ref[0] = jnp.concatenate(
            [jnp.broadcast_to(w / s, (w.shape[0], _L)) for w in ws], axis=1)

    col = jax.lax.broadcasted_iota(jnp.int32, adj.shape, 1)
    work = adj
    idxs, ws = [], []
    for k in range(_TOP_K):
        m = jnp.max(work, axis=1, keepdims=True)
        is_max = work == m
        idx = jnp.min(jnp.where(is_max, col, _N), axis=1, keepdims=True)
        if k < _TOP_K - 1:
            work = jnp.where(is_max, -1.0, work)
        idxs.append(idx)
        ws.append(m)
    cnt = jnp.sum((adj >= ws[-1]).astype(jnp.int32), axis=1, keepdims=True)
    ok = jnp.all(cnt == _TOP_K)

    @pl.when(ok)
    def _():
        _emit(idxs, ws)

    @pl.when(jnp.logical_not(ok))
    def _():
        w2 = adj
        eidxs, ews = [], []
        for _ in range(_TOP_K):
            m = jnp.max(w2, axis=1, keepdims=True)
            idx = jnp.min(jnp.where(w2 == m, col, _N), axis=1, keepdims=True)
            w2 = jnp.where(col == idx, -1.0, w2)
            eidxs.append(idx)
            ews.append(m)
        _emit(eidxs, ews)


def _sc_gather_body(xw_hbm, gidx_hbm, wexp_hbm, out_hbm,
                    idx_v, rows_v, w_v, acc_v, s0, s1, s2, s3):
    wid = lax.axis_index("s") * _NC + lax.axis_index("c")
    rbase = wid * _ROWS_W
    pltpu.sync_copy(gidx_hbm.at[pl.ds(rbase, _ROWS_W)], idx_v)
    sems = (s0, s1, s2, s3)
    copies = []
    for c in range(_NCHUNK):
        d = pl.ds(c * _ROWS_C, _ROWS_C)
        copies.append(
            pltpu.async_copy(xw_hbm.at[idx_v.at[d]], rows_v.at[d], sems[c]))
    pltpu.sync_copy(wexp_hbm.at[pl.ds(rbase * _L, _ROWS_W * _L)], w_v)

    def body(n, carry):
        r = n * _TOP_K
        w0 = w_v[pl.ds((r + 0) * _L, _L)]
        w1 = w_v[pl.ds((r + 1) * _L, _L)]
        w2 = w_v[pl.ds((r + 2) * _L, _L)]
        w3 = w_v[pl.ds((r + 3) * _L, _L)]
        for cc in range(_IN_DIM // _L):
            d = pl.ds(cc * _L, _L)
            acc = rows_v[r + 0, d] * w0
            acc = acc + rows_v[r + 1, d] * w1
            acc = acc + rows_v[r + 2, d] * w2
            acc = acc + rows_v[r + 3, d] * w3
            acc_v[n, d] = acc
        return carry

    for c in range(_NCHUNK):
        copies[c].wait()
        lax.fori_loop(c * _NODES_C, (c + 1) * _NODES_C, body, 0,
                      unroll=False)
    pltpu.sync_copy(acc_v, out_hbm.at[pl.ds(wid * _NODES_W, _NODES_W)])


_sc_gather = functools.partial(
    pl.kernel,
    out_type=jax.ShapeDtypeStruct((_B * _N, _OUT_DIM), jnp.float32),
    mesh=plsc.VectorSubcoreMesh(core_axis_name="c", subcore_axis_name="s"),
    scratch_types=[
        pltpu.VMEM((_ROWS_W,), jnp.int32),
        pltpu.VMEM((_ROWS_W, _OUT_DIM), jnp.float32),
        pltpu.VMEM((_ROWS_W * _L,), jnp.float32),
        pltpu.VMEM((_NODES_W, _OUT_DIM), jnp.float32),
        pltpu.SemaphoreType.DMA,
        pltpu.SemaphoreType.DMA,
        pltpu.SemaphoreType.DMA,
        pltpu.SemaphoreType.DMA,
    ],
)(_sc_gather_body)


@jax.jit
def kernel(x, pos, W, b):
    pos8 = jnp.pad(pos, ((0, 0), (0, 0), (0, 5)))
    pos_t = jnp.transpose(pos8, (0, 2, 1))
    wt = W.T
    b2 = b.reshape(1, _OUT_DIM)

    gidx, wexp, xw = pl.pallas_call(
        _topk_body,
        grid=(_B, _N // _BR),
        in_specs=[
            pl.BlockSpec((1, _BR, 8), lambda bi, i: (bi, i, 0)),
            pl.BlockSpec((1, 8, _N), lambda bi, i: (bi, 0, 0)),
            pl.BlockSpec((1, _BR, _IN_DIM), lambda bi, i: (bi, i, 0)),
            pl.BlockSpec((_IN_DIM, _OUT_DIM), lambda bi, i: (0, 0)),
            pl.BlockSpec((1, _OUT_DIM), lambda bi, i: (0, 0)),
        ],
        out_specs=[
            pl.BlockSpec((1, _BR, _TOP_K), lambda bi, i: (bi, i, 0)),
            pl.BlockSpec((1, _BR, _TOP_K * _L), lambda bi, i: (bi, i, 0)),
            pl.BlockSpec((1, _BR, _OUT_DIM), lambda bi, i: (bi, i, 0)),
        ],
        out_shape=[
            jax.ShapeDtypeStruct((_B, _N, _TOP_K), jnp.int32),
            jax.ShapeDtypeStruct((_B, _N, _TOP_K * _L), jnp.float32),
            jax.ShapeDtypeStruct((_B, _N, _OUT_DIM), jnp.float32),
        ],
    )(pos8, pos_t, x, wt, b2)

    xw2d = xw.reshape(_B * _N, _OUT_DIM)
    gidx_flat = gidx.reshape(_B * _N * _TOP_K)
    wexp_flat = wexp.reshape(_B * _N * _TOP_K * _L)
    out = _sc_gather(xw2d, gidx_flat, wexp_flat)
    return out.reshape(_B, _N, _OUT_DIM)


# speedup vs baseline: 1.0370x; 1.0370x over previous
import functools

import jax
import jax.numpy as jnp
from jax import lax
from jax.experimental import pallas as pl
from jax.experimental.pallas import tpu as pltpu
from jax.experimental.pallas import tpu_sc as plsc

_B, _N, _IN_DIM, _OUT_DIM = 2, 2048, 128, 128
_TOP_K = 4
_BR = 256

_NC, _NS, _L = 2, 16, 16
_NW = _NC * _NS
_NODES_W = (_B * _N) // _NW
_ROWS_W = _NODES_W * _TOP_K
_NCHUNK = 4
_NODES_C = _NODES_W // _NCHUNK
_ROWS_C = _ROWS_W // _NCHUNK


def _topk_body(pos_blk, pos_t, x_blk, wt_ref, b_ref,
               gidx_ref, wexp_ref, xw_ref):
    pi = pos_blk[0]
    pj = pos_t[0]

    xw_ref[0] = jnp.dot(x_blk[0], wt_ref[...],
                        preferred_element_type=jnp.float32) + b_ref[...]

    dx = pi[:, 0:1] - pj[0:1, :]
    dy = pi[:, 1:2] - pj[1:2, :]
    dz = pi[:, 2:3] - pj[2:3, :]
    dsq = dx * dx + dy * dy + dz * dz
    dist = jnp.sqrt(dsq + 1e-8)
    adj = jnp.exp(-(dist * dist) * 0.5)

    col = jax.lax.broadcasted_iota(jnp.int32, adj.shape, 1)
    work = adj
    idxs, ws = [], []
    for _ in range(_TOP_K):
        m = jnp.max(work, axis=1, keepdims=True)
        idx = jnp.min(jnp.where(work == m, col, _N), axis=1, keepdims=True)
        work = jnp.where(col == idx, -1.0, work)
        idxs.append(idx)
        ws.append(m)

    s = ((ws[0] + ws[1]) + ws[2]) + ws[3] + 1e-8
    boff = pl.program_id(0) * _N
    gidx_ref[0] = jnp.concatenate([i + boff for i in idxs], axis=1)
    wexp_ref[0] = jnp.concatenate(
        [jnp.broadcast_to(w / s, (w.shape[0], _L)) for w in ws], axis=1)


def _sc_gather_body(xw_hbm, gidx_hbm, wexp_hbm, out_hbm,
                    idx_v, rows_v, w_v, acc_v, s0, s1, s2, s3):
    wid = lax.axis_index("s") * _NC + lax.axis_index("c")
    rbase = wid * _ROWS_W
    pltpu.sync_copy(gidx_hbm.at[pl.ds(rbase, _ROWS_W)], idx_v)
    sems = (s0, s1, s2, s3)
    copies = []
    for c in range(_NCHUNK):
        d = pl.ds(c * _ROWS_C, _ROWS_C)
        copies.append(
            pltpu.async_copy(xw_hbm.at[idx_v.at[d]], rows_v.at[d], sems[c]))
    pltpu.sync_copy(wexp_hbm.at[pl.ds(rbase * _L, _ROWS_W * _L)], w_v)

    def body(n, carry):
        r = n * _TOP_K
        w0 = w_v[pl.ds((r + 0) * _L, _L)]
        w1 = w_v[pl.ds((r + 1) * _L, _L)]
        w2 = w_v[pl.ds((r + 2) * _L, _L)]
        w3 = w_v[pl.ds((r + 3) * _L, _L)]
        for cc in range(_IN_DIM // _L):
            d = pl.ds(cc * _L, _L)
            acc = rows_v[r + 0, d] * w0
            acc = acc + rows_v[r + 1, d] * w1
            acc = acc + rows_v[r + 2, d] * w2
            acc = acc + rows_v[r + 3, d] * w3
            acc_v[n, d] = acc
        return carry

    for c in range(_NCHUNK):
        copies[c].wait()
        lax.fori_loop(c * _NODES_C, (c + 1) * _NODES_C, body, 0,
                      unroll=False)
    pltpu.sync_copy(acc_v, out_hbm.at[pl.ds(wid * _NODES_W, _NODES_W)])


_sc_gather = functools.partial(
    pl.kernel,
    out_type=jax.ShapeDtypeStruct((_B * _N, _OUT_DIM), jnp.float32),
    mesh=plsc.VectorSubcoreMesh(core_axis_name="c", subcore_axis_name="s"),
    scratch_types=[
        pltpu.VMEM((_ROWS_W,), jnp.int32),
        pltpu.VMEM((_ROWS_W, _OUT_DIM), jnp.float32),
        pltpu.VMEM((_ROWS_W * _L,), jnp.float32),
        pltpu.VMEM((_NODES_W, _OUT_DIM), jnp.float32),
        pltpu.SemaphoreType.DMA,
        pltpu.SemaphoreType.DMA,
        pltpu.SemaphoreType.DMA,
        pltpu.SemaphoreType.DMA,
    ],
)(_sc_gather_body)


@jax.jit
def kernel(x, pos, W, b):
    pos8 = jnp.pad(pos, ((0, 0), (0, 0), (0, 5)))
    pos_t = jnp.transpose(pos8, (0, 2, 1))
    wt = W.T
    b2 = b.reshape(1, _OUT_DIM)

    gidx, wexp, xw = pl.pallas_call(
        _topk_body,
        grid=(_B, _N // _BR),
        in_specs=[
            pl.BlockSpec((1, _BR, 8), lambda bi, i: (bi, i, 0)),
            pl.BlockSpec((1, 8, _N), lambda bi, i: (bi, 0, 0)),
            pl.BlockSpec((1, _BR, _IN_DIM), lambda bi, i: (bi, i, 0)),
            pl.BlockSpec((_IN_DIM, _OUT_DIM), lambda bi, i: (0, 0)),
            pl.BlockSpec((1, _OUT_DIM), lambda bi, i: (0, 0)),
        ],
        out_specs=[
            pl.BlockSpec((1, _BR, _TOP_K), lambda bi, i: (bi, i, 0)),
            pl.BlockSpec((1, _BR, _TOP_K * _L), lambda bi, i: (bi, i, 0)),
            pl.BlockSpec((1, _BR, _OUT_DIM), lambda bi, i: (bi, i, 0)),
        ],
        out_shape=[
            jax.ShapeDtypeStruct((_B, _N, _TOP_K), jnp.int32),
            jax.ShapeDtypeStruct((_B, _N, _TOP_K * _L), jnp.float32),
            jax.ShapeDtypeStruct((_B, _N, _OUT_DIM), jnp.float32),
        ],
    )(pos8, pos_t, x, wt, b2)

    xw2d = xw.reshape(_B * _N, _OUT_DIM)
    gidx_flat = gidx.reshape(_B * _N * _TOP_K)
    wexp_flat = wexp.reshape(_B * _N * _TOP_K * _L)
    out = _sc_gather(xw2d, gidx_flat, wexp_flat)
    return out.reshape(_B, _N, _OUT_DIM)


# speedup vs baseline: 1.0773x; 1.0389x over previous
import functools

import jax
import jax.numpy as jnp
from jax import lax
from jax.experimental import pallas as pl
from jax.experimental.pallas import tpu as pltpu
from jax.experimental.pallas import tpu_sc as plsc

_B, _N, _IN_DIM, _OUT_DIM = 2, 2048, 128, 128
_TOP_K = 4
_BR = 512

_NC, _NS, _L = 2, 16, 16
_NW = _NC * _NS
_NODES_W = (_B * _N) // _NW
_ROWS_W = _NODES_W * _TOP_K
_NCHUNK = 4
_NODES_C = _NODES_W // _NCHUNK
_ROWS_C = _ROWS_W // _NCHUNK


def _topk_body(pos_blk, pos_t, x_blk, wt_ref, b_ref,
               gidx_ref, wexp_ref, xw_ref):
    pi = pos_blk[0]
    pj = pos_t[0]

    xw_ref[0] = jnp.dot(x_blk[0], wt_ref[...],
                        preferred_element_type=jnp.float32) + b_ref[...]

    dx = pi[:, 0:1] - pj[0:1, :]
    dy = pi[:, 1:2] - pj[1:2, :]
    dz = pi[:, 2:3] - pj[2:3, :]
    dsq = dx * dx + dy * dy + dz * dz
    dist = jnp.sqrt(dsq + 1e-8)
    adj = jnp.exp(-(dist * dist) * 0.5)

    col = jax.lax.broadcasted_iota(jnp.int32, adj.shape, 1)
    work = adj
    idxs, ws = [], []
    for _ in range(_TOP_K):
        m = jnp.max(work, axis=1, keepdims=True)
        idx = jnp.min(jnp.where(work == m, col, _N), axis=1, keepdims=True)
        work = jnp.where(col == idx, -1.0, work)
        idxs.append(idx)
        ws.append(m)

    s = ((ws[0] + ws[1]) + ws[2]) + ws[3] + 1e-8
    boff = pl.program_id(0) * _N
    gidx_ref[0] = jnp.concatenate([i + boff for i in idxs], axis=1)
    wexp_ref[0] = jnp.concatenate(
        [jnp.broadcast_to(w / s, (w.shape[0], _L)) for w in ws], axis=1)


def _sc_gather_body(xw_hbm, gidx_hbm, wexp_hbm, out_hbm,
                    idx_v, rows_v, w_v, acc_v, s0, s1, s2, s3):
    wid = lax.axis_index("s") * _NC + lax.axis_index("c")
    rbase = wid * _ROWS_W
    pltpu.sync_copy(gidx_hbm.at[pl.ds(rbase, _ROWS_W)], idx_v)
    sems = (s0, s1, s2, s3)
    copies = []
    for c in range(_NCHUNK):
        d = pl.ds(c * _ROWS_C, _ROWS_C)
        copies.append(
            pltpu.async_copy(xw_hbm.at[idx_v.at[d]], rows_v.at[d], sems[c]))
    pltpu.sync_copy(wexp_hbm.at[pl.ds(rbase * _L, _ROWS_W * _L)], w_v)

    def body(n, carry):
        r = n * _TOP_K
        w0 = w_v[pl.ds((r + 0) * _L, _L)]
        w1 = w_v[pl.ds((r + 1) * _L, _L)]
        w2 = w_v[pl.ds((r + 2) * _L, _L)]
        w3 = w_v[pl.ds((r + 3) * _L, _L)]
        for cc in range(_IN_DIM // _L):
            d = pl.ds(cc * _L, _L)
            acc = rows_v[r + 0, d] * w0
            acc = acc + rows_v[r + 1, d] * w1
            acc = acc + rows_v[r + 2, d] * w2
            acc = acc + rows_v[r + 3, d] * w3
            acc_v[n, d] = acc
        return carry

    for c in range(_NCHUNK):
        copies[c].wait()
        lax.fori_loop(c * _NODES_C, (c + 1) * _NODES_C, body, 0,
                      unroll=False)
    pltpu.sync_copy(acc_v, out_hbm.at[pl.ds(wid * _NODES_W, _NODES_W)])


_sc_gather = functools.partial(
    pl.kernel,
    out_type=jax.ShapeDtypeStruct((_B * _N, _OUT_DIM), jnp.float32),
    mesh=plsc.VectorSubcoreMesh(core_axis_name="c", subcore_axis_name="s"),
    scratch_types=[
        pltpu.VMEM((_ROWS_W,), jnp.int32),
        pltpu.VMEM((_ROWS_W, _OUT_DIM), jnp.float32),
        pltpu.VMEM((_ROWS_W * _L,), jnp.float32),
        pltpu.VMEM((_NODES_W, _OUT_DIM), jnp.float32),
        pltpu.SemaphoreType.DMA,
        pltpu.SemaphoreType.DMA,
        pltpu.SemaphoreType.DMA,
        pltpu.SemaphoreType.DMA,
    ],
)(_sc_gather_body)


@jax.jit
def kernel(x, pos, W, b):
    pos8 = jnp.pad(pos, ((0, 0), (0, 0), (0, 5)))
    pos_t = jnp.transpose(pos8, (0, 2, 1))
    wt = W.T
    b2 = b.reshape(1, _OUT_DIM)

    gidx, wexp, xw = pl.pallas_call(
        _topk_body,
        grid=(_B, _N // _BR),
        in_specs=[
            pl.BlockSpec((1, _BR, 8), lambda bi, i: (bi, i, 0)),
            pl.BlockSpec((1, 8, _N), lambda bi, i: (bi, 0, 0)),
            pl.BlockSpec((1, _BR, _IN_DIM), lambda bi, i: (bi, i, 0)),
            pl.BlockSpec((_IN_DIM, _OUT_DIM), lambda bi, i: (0, 0)),
            pl.BlockSpec((1, _OUT_DIM), lambda bi, i: (0, 0)),
        ],
        out_specs=[
            pl.BlockSpec((1, _BR, _TOP_K), lambda bi, i: (bi, i, 0)),
            pl.BlockSpec((1, _BR, _TOP_K * _L), lambda bi, i: (bi, i, 0)),
            pl.BlockSpec((1, _BR, _OUT_DIM), lambda bi, i: (bi, i, 0)),
        ],
        out_shape=[
            jax.ShapeDtypeStruct((_B, _N, _TOP_K), jnp.int32),
            jax.ShapeDtypeStruct((_B, _N, _TOP_K * _L), jnp.float32),
            jax.ShapeDtypeStruct((_B, _N, _OUT_DIM), jnp.float32),
        ],
    )(pos8, pos_t, x, wt, b2)

    xw2d = xw.reshape(_B * _N, _OUT_DIM)
    gidx_flat = gidx.reshape(_B * _N * _TOP_K)
    wexp_flat = wexp.reshape(_B * _N * _TOP_K * _L)
    out = _sc_gather(xw2d, gidx_flat, wexp_flat)
    return out.reshape(_B, _N, _OUT_DIM)
